# baseline (speedup 1.0000x reference)
import jax
import jax.numpy as jnp
from jax.experimental import pallas as pl


def _bcast_kernel(n_sblk, block_s, pos_ref, o_ref):
    i = pl.program_id(0)
    s = (i % n_sblk) * block_s
    o_ref[...] = pos_ref[pl.ds(s, block_s), :]


def kernel(x, pos_embeddings):
    B, S = x.shape
    D = pos_embeddings.shape[1]
    block_s = 2048
    n_sblk = S // block_s
    import functools
    out_flat = pl.pallas_call(
        functools.partial(_bcast_kernel, n_sblk, block_s),
        grid=(B * n_sblk,),
        in_specs=[pl.BlockSpec((S, D), lambda i: (0, 0))],
        out_specs=pl.BlockSpec((block_s, D), lambda i: (i, 0)),
        out_shape=jax.ShapeDtypeStruct((B * S, D), pos_embeddings.dtype),
    )(pos_embeddings)
    return out_flat.reshape(B, S, D)
